# baseline (speedup 1.0000x reference)
import functools

import jax
import jax.numpy as jnp
from jax import lax
from jax.experimental import pallas as pl
from jax.experimental.pallas import tpu as pltpu
from jax.experimental.pallas import tpu_sc as plsc

N = 10000
E = 320000
D = 128
C = 16

NC = 2
NS = 16
NW = NC * NS
EPW = E // NW
K = 80
NCHUNK = EPW // K
NG = 5
G = NCHUNK // NG
N2 = 10240
N2PT = N2 // NS


def _sc_mesh():
    return plsc.VectorSubcoreMesh(core_axis_name="c", subcore_axis_name="s")


def _agg_deg_body(feat, srcb, dstb, z2, z1,
                  agg_out, deg_out,
                  src_v, dst_v, rows0, rows1, ones_v, sem0, sem1, ssem0, ssem1, dsem,
                  acc, dega):
    c = lax.axis_index("c")
    s = lax.axis_index("s")
    w = c * NS + s

    pltpu.sync_copy(z2.at[pl.ds(s * N2PT, N2PT)], acc.at[pl.ds(s * N2PT, N2PT)])
    pltpu.sync_copy(z1.at[pl.ds(s * N2PT, N2PT)], dega.at[pl.ds(s * N2PT, N2PT)])

    for j in range(K // 16):
        ones_v[pl.ds(j * 16, 16)] = jnp.ones((16,), jnp.float32)

    plsc.subcore_barrier()

    def group(g, carry):
        pltpu.sync_copy(srcb.at[w, g], src_v)
        pltpu.sync_copy(dstb.at[w, g], dst_v)
        pltpu.async_copy(feat.at[src_v.at[0]], rows0, sem0)
        pltpu.async_copy(feat.at[src_v.at[1]], rows1, sem1)

        def pair(j, c2):
            i0 = 2 * j
            i1 = i0 + 1
            pltpu.make_async_copy(feat.at[src_v.at[i0]], rows0, sem0).wait()

            @pl.when(j > 0)
            def _():
                pltpu.make_async_copy(ones_v, dega.at[dst_v.at[0]], dsem).wait()
                pltpu.make_async_copy(ones_v, dega.at[dst_v.at[0]], dsem).wait()

            pltpu.async_copy(rows0, acc.at[dst_v.at[i0]], ssem0, add=True)
            pltpu.async_copy(ones_v, dega.at[dst_v.at[i0]], dsem, add=True)
            pltpu.make_async_copy(feat.at[src_v.at[i1]], rows1, sem1).wait()
            pltpu.async_copy(rows1, acc.at[dst_v.at[i1]], ssem1, add=True)
            pltpu.async_copy(ones_v, dega.at[dst_v.at[i1]], dsem, add=True)
            pltpu.make_async_copy(rows0, acc.at[dst_v.at[0]], ssem0).wait()
            pltpu.async_copy(feat.at[src_v.at[i0 + 2]], rows0, sem0)
            pltpu.make_async_copy(rows1, acc.at[dst_v.at[0]], ssem1).wait()
            pltpu.async_copy(feat.at[src_v.at[i1 + 2]], rows1, sem1)
            return c2

        lax.fori_loop(0, (G - 3) // 2, pair, 0)

        pltpu.make_async_copy(feat.at[src_v.at[G - 3]], rows0, sem0).wait()
        pltpu.async_copy(rows0, acc.at[dst_v.at[G - 3]], ssem0, add=True)
        pltpu.async_copy(ones_v, dega.at[dst_v.at[G - 3]], dsem, add=True)
        pltpu.make_async_copy(rows0, acc.at[dst_v.at[0]], ssem0).wait()
        pltpu.async_copy(feat.at[src_v.at[G - 1]], rows0, sem0)
        pltpu.make_async_copy(feat.at[src_v.at[G - 2]], rows1, sem1).wait()
        pltpu.async_copy(rows1, acc.at[dst_v.at[G - 2]], ssem1, add=True)
        pltpu.async_copy(ones_v, dega.at[dst_v.at[G - 2]], dsem, add=True)
        pltpu.make_async_copy(feat.at[src_v.at[G - 1]], rows0, sem0).wait()
        pltpu.async_copy(rows0, acc.at[dst_v.at[G - 1]], ssem0, add=True)
        pltpu.async_copy(ones_v, dega.at[dst_v.at[G - 1]], dsem, add=True)

        pltpu.make_async_copy(rows0, acc.at[dst_v.at[0]], ssem0).wait()
        pltpu.make_async_copy(rows1, acc.at[dst_v.at[0]], ssem1).wait()
        for _ in range(5):
            pltpu.make_async_copy(ones_v, dega.at[dst_v.at[0]], dsem).wait()
        return carry

    lax.fori_loop(0, NG, group, 0)

    plsc.subcore_barrier()

    pltpu.sync_copy(acc.at[pl.ds(s * N2PT, N2PT)], agg_out.at[c, pl.ds(s * N2PT, N2PT)])
    pltpu.sync_copy(dega.at[pl.ds(s * N2PT, N2PT)], deg_out.at[c, pl.ds(s * N2PT, N2PT)])


def _agg16_body(feat, srcb, dstb, z2,
                agg_out,
                src_v, dst_v, rows0, rows1, sem0, sem1, ssem0, ssem1,
                acc):
    c = lax.axis_index("c")
    s = lax.axis_index("s")
    w = c * NS + s

    pltpu.sync_copy(z2.at[pl.ds(s * N2PT, N2PT)], acc.at[pl.ds(s * N2PT, N2PT)])

    plsc.subcore_barrier()

    def group(g, carry):
        pltpu.sync_copy(srcb.at[w, g], src_v)
        pltpu.sync_copy(dstb.at[w, g], dst_v)
        pltpu.async_copy(feat.at[src_v.at[0]], rows0, sem0)
        pltpu.async_copy(feat.at[src_v.at[1]], rows1, sem1)

        def pair(j, c2):
            i0 = 2 * j
            i1 = i0 + 1
            pltpu.make_async_copy(feat.at[src_v.at[i0]], rows0, sem0).wait()
            pltpu.async_copy(rows0, acc.at[dst_v.at[i0]], ssem0, add=True)
            pltpu.make_async_copy(feat.at[src_v.at[i1]], rows1, sem1).wait()
            pltpu.async_copy(rows1, acc.at[dst_v.at[i1]], ssem1, add=True)
            pltpu.make_async_copy(rows0, acc.at[dst_v.at[0]], ssem0).wait()
            pltpu.async_copy(feat.at[src_v.at[i0 + 2]], rows0, sem0)
            pltpu.make_async_copy(rows1, acc.at[dst_v.at[0]], ssem1).wait()
            pltpu.async_copy(feat.at[src_v.at[i1 + 2]], rows1, sem1)
            return c2

        lax.fori_loop(0, (G - 3) // 2, pair, 0)

        pltpu.make_async_copy(feat.at[src_v.at[G - 3]], rows0, sem0).wait()
        pltpu.async_copy(rows0, acc.at[dst_v.at[G - 3]], ssem0, add=True)
        pltpu.make_async_copy(rows0, acc.at[dst_v.at[0]], ssem0).wait()
        pltpu.async_copy(feat.at[src_v.at[G - 1]], rows0, sem0)
        pltpu.make_async_copy(feat.at[src_v.at[G - 2]], rows1, sem1).wait()
        pltpu.async_copy(rows1, acc.at[dst_v.at[G - 2]], ssem1, add=True)
        pltpu.make_async_copy(feat.at[src_v.at[G - 1]], rows0, sem0).wait()
        pltpu.async_copy(rows0, acc.at[dst_v.at[G - 1]], ssem0, add=True)

        pltpu.make_async_copy(rows0, acc.at[dst_v.at[0]], ssem0).wait()
        pltpu.make_async_copy(rows1, acc.at[dst_v.at[0]], ssem1).wait()
        return carry

    lax.fori_loop(0, NG, group, 0)

    plsc.subcore_barrier()

    pltpu.sync_copy(acc.at[pl.ds(s * N2PT, N2PT)], agg_out.at[c, pl.ds(s * N2PT, N2PT)])


def _sc_agg_deg(x, srcb, dstb):
    f = pl.kernel(
        _agg_deg_body,
        out_type=[jax.ShapeDtypeStruct((NC, N2, D), jnp.float32),
                  jax.ShapeDtypeStruct((NC, N2), jnp.float32)],
        mesh=_sc_mesh(),
        scratch_types=[
            pltpu.VMEM((G, K), jnp.int32),
            pltpu.VMEM((G, K), jnp.int32),
            pltpu.VMEM((K, D), jnp.float32),
            pltpu.VMEM((K, D), jnp.float32),
            pltpu.VMEM((K,), jnp.float32),
            pltpu.SemaphoreType.DMA,
            pltpu.SemaphoreType.DMA,
            pltpu.SemaphoreType.DMA,
            pltpu.SemaphoreType.DMA,
            pltpu.SemaphoreType.DMA,
            pltpu.VMEM_SHARED((N2, D), jnp.float32),
            pltpu.VMEM_SHARED((N2,), jnp.float32),
        ],
    )
    z2 = jnp.zeros((N2, D), jnp.float32)
    z1 = jnp.zeros((N2,), jnp.float32)
    return f(x, srcb, dstb, z2, z1)


def _sc_agg16(p, srcb, dstb):
    f = pl.kernel(
        _agg16_body,
        out_type=jax.ShapeDtypeStruct((NC, N2, C), jnp.float32),
        mesh=_sc_mesh(),
        compiler_params=pltpu.CompilerParams(use_tc_tiling_on_sc=False),
        scratch_types=[
            pltpu.VMEM((G, K), jnp.int32),
            pltpu.VMEM((G, K), jnp.int32),
            pltpu.VMEM((K, C), jnp.float32),
            pltpu.VMEM((K, C), jnp.float32),
            pltpu.SemaphoreType.DMA,
            pltpu.SemaphoreType.DMA,
            pltpu.SemaphoreType.DMA,
            pltpu.SemaphoreType.DMA,
            pltpu.VMEM_SHARED((N2, C), jnp.float32),
        ],
    )
    z2 = jnp.zeros((N2, C), jnp.float32)
    return f(p, srcb, dstb, z2)


BN = 400
_PREC = lax.Precision.HIGHEST
_DN = (((1,), (1,)), ((), ()))


def _tc1_body(aggp, degt, x, W1l, W1r, b1, W2l, W2r, b2, p_out, q_out):
    agg = aggp[0] + aggp[1]
    deg = degt[:, 0:1] + degt[:, 1:2]
    rdeg = 1.0 / jnp.maximum(deg, 1.0)
    mean = agg * rdeg
    h = (lax.dot_general(mean, W1l[...], _DN, precision=_PREC)
         + lax.dot_general(x[...], W1r[...], _DN, precision=_PREC)
         + b1[...])
    h = jnp.maximum(h, 0.0)
    p_out[...] = lax.dot_general(h, W2l[...], _DN, precision=_PREC)
    q_out[...] = lax.dot_general(h, W2r[...], _DN, precision=_PREC) + b2[...]


def _tc_layer1(aggp, degt, x, W1l, b1, W1r, W2l, b2, W2r):
    grid = (N // BN,)
    return pl.pallas_call(
        _tc1_body,
        grid=grid,
        in_specs=[
            pl.BlockSpec((NC, BN, D), lambda i: (0, i, 0)),
            pl.BlockSpec((BN, NC), lambda i: (i, 0)),
            pl.BlockSpec((BN, D), lambda i: (i, 0)),
            pl.BlockSpec((D, D), lambda i: (0, 0)),
            pl.BlockSpec((D, D), lambda i: (0, 0)),
            pl.BlockSpec((1, D), lambda i: (0, 0)),
            pl.BlockSpec((C, D), lambda i: (0, 0)),
            pl.BlockSpec((C, D), lambda i: (0, 0)),
            pl.BlockSpec((1, C), lambda i: (0, 0)),
        ],
        out_specs=[
            pl.BlockSpec((BN, C), lambda i: (i, 0)),
            pl.BlockSpec((BN, C), lambda i: (i, 0)),
        ],
        out_shape=[jax.ShapeDtypeStruct((N, C), jnp.float32),
                   jax.ShapeDtypeStruct((N, C), jnp.float32)],
    )(aggp, degt, x, W1l, W1r, b1.reshape(1, D), W2l, W2r, b2.reshape(1, C))


def _tc2_body(aggp, degt, q, out):
    agg = aggp[0] + aggp[1]
    deg = degt[:, 0:1] + degt[:, 1:2]
    rdeg = 1.0 / jnp.maximum(deg, 1.0)
    z = agg * rdeg + q[...]
    m = jnp.max(z, axis=1, keepdims=True)
    zs = z - m
    out[...] = zs - jnp.log(jnp.sum(jnp.exp(zs), axis=1, keepdims=True))


def _tc_layer2(aggp, degt, q):
    grid = (N // BN,)
    return pl.pallas_call(
        _tc2_body,
        grid=grid,
        in_specs=[
            pl.BlockSpec((NC, BN, C), lambda i: (0, i, 0)),
            pl.BlockSpec((BN, NC), lambda i: (i, 0)),
            pl.BlockSpec((BN, C), lambda i: (i, 0)),
        ],
        out_specs=pl.BlockSpec((BN, C), lambda i: (i, 0)),
        out_shape=jax.ShapeDtypeStruct((N, C), jnp.float32),
    )(aggp, degt, q)


def kernel(x, edge_index, W1l, b1, W1r, W2l, b2, W2r):
    srcb = edge_index[0].reshape(NW, NG, G, K)
    dstb = edge_index[1].reshape(NW, NG, G, K)

    agg1, degp = _sc_agg_deg(x, srcb, dstb)
    degt = degp.T
    p, q = _tc_layer1(agg1, degt, x, W1l, b1, W1r, W2l, b2, W2r)
    agg2 = _sc_agg16(p, srcb, dstb)
    return _tc_layer2(agg2, degt, q)


# speedup vs baseline: 1.1077x; 1.1077x over previous
import functools

import jax
import jax.numpy as jnp
from jax import lax
from jax.experimental import pallas as pl
from jax.experimental.pallas import tpu as pltpu
from jax.experimental.pallas import tpu_sc as plsc

N = 10000
E = 320000
D = 128
C = 16

NC = 2
NS = 16
NW = NC * NS
EPW = E // NW
K = 100
NCHUNK = EPW // K
NG = 4
G = NCHUNK // NG
N2 = 10240
N2PT = N2 // NS


def _sc_mesh():
    return plsc.VectorSubcoreMesh(core_axis_name="c", subcore_axis_name="s")


def _agg_deg_body(feat, srcb, dstb, z2, z1,
                  agg_out, deg_out,
                  src_v, dst_v, rows0, rows1, ones_v, sem0, sem1,
                  acc, dega):
    c = lax.axis_index("c")
    s = lax.axis_index("s")
    w = c * NS + s

    pltpu.sync_copy(z2.at[pl.ds(s * N2PT, N2PT)], acc.at[pl.ds(s * N2PT, N2PT)])
    pltpu.sync_copy(z1.at[pl.ds(s * N2PT, N2PT)], dega.at[pl.ds(s * N2PT, N2PT)])

    for j in range(K // 16):
        ones_v[pl.ds(j * 16, 16)] = jnp.ones((16,), jnp.float32)
    if K % 16:
        ones_v[pl.ds(K - 16, 16)] = jnp.ones((16,), jnp.float32)

    plsc.subcore_barrier()

    def group(g, carry):
        pltpu.sync_copy(srcb.at[w, g], src_v)
        pltpu.sync_copy(dstb.at[w, g], dst_v)
        pltpu.async_copy(feat.at[src_v.at[0]], rows0, sem0)

        def pair(j, c2):
            i0 = 2 * j
            i1 = i0 + 1
            pltpu.async_copy(feat.at[src_v.at[i1]], rows1, sem1)
            pltpu.make_async_copy(feat.at[src_v.at[i0]], rows0, sem0).wait()
            pltpu.sync_copy(rows0, acc.at[dst_v.at[i0]], add=True)
            pltpu.sync_copy(ones_v, dega.at[dst_v.at[i0]], add=True)
            pltpu.async_copy(feat.at[src_v.at[i0 + 2]], rows0, sem0)
            pltpu.make_async_copy(feat.at[src_v.at[i1]], rows1, sem1).wait()
            pltpu.sync_copy(rows1, acc.at[dst_v.at[i1]], add=True)
            pltpu.sync_copy(ones_v, dega.at[dst_v.at[i1]], add=True)
            return c2

        lax.fori_loop(0, (G - 1) // 2, pair, 0)

        last = G - 1
        pltpu.make_async_copy(feat.at[src_v.at[last]], rows0, sem0).wait()
        pltpu.sync_copy(rows0, acc.at[dst_v.at[last]], add=True)
        pltpu.sync_copy(ones_v, dega.at[dst_v.at[last]], add=True)
        return carry

    lax.fori_loop(0, NG, group, 0)

    plsc.subcore_barrier()

    pltpu.sync_copy(acc.at[pl.ds(s * N2PT, N2PT)], agg_out.at[c, pl.ds(s * N2PT, N2PT)])
    pltpu.sync_copy(dega.at[pl.ds(s * N2PT, N2PT)], deg_out.at[c, pl.ds(s * N2PT, N2PT)])


def _agg16_body(feat, srcb, dstb, z2,
                agg_out,
                src_v, dst_v, rows0, rows1, sem0, sem1,
                acc):
    c = lax.axis_index("c")
    s = lax.axis_index("s")
    w = c * NS + s

    pltpu.sync_copy(z2.at[pl.ds(s * N2PT, N2PT)], acc.at[pl.ds(s * N2PT, N2PT)])

    plsc.subcore_barrier()

    def group(g, carry):
        pltpu.sync_copy(srcb.at[w, g], src_v)
        pltpu.sync_copy(dstb.at[w, g], dst_v)
        pltpu.async_copy(feat.at[src_v.at[0]], rows0, sem0)

        def pair(j, c2):
            i0 = 2 * j
            i1 = i0 + 1
            pltpu.async_copy(feat.at[src_v.at[i1]], rows1, sem1)
            pltpu.make_async_copy(feat.at[src_v.at[i0]], rows0, sem0).wait()
            pltpu.sync_copy(rows0, acc.at[dst_v.at[i0]], add=True)
            pltpu.async_copy(feat.at[src_v.at[i0 + 2]], rows0, sem0)
            pltpu.make_async_copy(feat.at[src_v.at[i1]], rows1, sem1).wait()
            pltpu.sync_copy(rows1, acc.at[dst_v.at[i1]], add=True)
            return c2

        lax.fori_loop(0, (G - 1) // 2, pair, 0)

        last = G - 1
        pltpu.make_async_copy(feat.at[src_v.at[last]], rows0, sem0).wait()
        pltpu.sync_copy(rows0, acc.at[dst_v.at[last]], add=True)
        return carry

    lax.fori_loop(0, NG, group, 0)

    plsc.subcore_barrier()

    pltpu.sync_copy(acc.at[pl.ds(s * N2PT, N2PT)], agg_out.at[c, pl.ds(s * N2PT, N2PT)])


def _sc_agg_deg(x, srcb, dstb):
    f = pl.kernel(
        _agg_deg_body,
        out_type=[jax.ShapeDtypeStruct((NC, N2, D), jnp.float32),
                  jax.ShapeDtypeStruct((NC, N2), jnp.float32)],
        mesh=_sc_mesh(),
        scratch_types=[
            pltpu.VMEM((G, K), jnp.int32),
            pltpu.VMEM((G, K), jnp.int32),
            pltpu.VMEM((K, D), jnp.float32),
            pltpu.VMEM((K, D), jnp.float32),
            pltpu.VMEM((K,), jnp.float32),
            pltpu.SemaphoreType.DMA,
            pltpu.SemaphoreType.DMA,
            pltpu.VMEM_SHARED((N2, D), jnp.float32),
            pltpu.VMEM_SHARED((N2,), jnp.float32),
        ],
    )
    z2 = jnp.zeros((N2, D), jnp.float32)
    z1 = jnp.zeros((N2,), jnp.float32)
    return f(x, srcb, dstb, z2, z1)


def _sc_agg16(p, srcb, dstb):
    f = pl.kernel(
        _agg16_body,
        out_type=jax.ShapeDtypeStruct((NC, N2, C), jnp.float32),
        mesh=_sc_mesh(),
        compiler_params=pltpu.CompilerParams(use_tc_tiling_on_sc=False),
        scratch_types=[
            pltpu.VMEM((G, K), jnp.int32),
            pltpu.VMEM((G, K), jnp.int32),
            pltpu.VMEM((K, C), jnp.float32),
            pltpu.VMEM((K, C), jnp.float32),
            pltpu.SemaphoreType.DMA,
            pltpu.SemaphoreType.DMA,
            pltpu.VMEM_SHARED((N2, C), jnp.float32),
        ],
    )
    z2 = jnp.zeros((N2, C), jnp.float32)
    return f(p, srcb, dstb, z2)


BN = 400
_PREC = lax.Precision.HIGHEST
_DN = (((1,), (1,)), ((), ()))


def _tc1_body(aggp, degt, x, W1l, W1r, b1, W2l, W2r, b2, p_out, q_out):
    agg = aggp[0] + aggp[1]
    deg = degt[:, 0:1] + degt[:, 1:2]
    rdeg = 1.0 / jnp.maximum(deg, 1.0)
    mean = agg * rdeg
    h = (lax.dot_general(mean, W1l[...], _DN, precision=_PREC)
         + lax.dot_general(x[...], W1r[...], _DN, precision=_PREC)
         + b1[...])
    h = jnp.maximum(h, 0.0)
    p_out[...] = lax.dot_general(h, W2l[...], _DN, precision=_PREC)
    q_out[...] = lax.dot_general(h, W2r[...], _DN, precision=_PREC) + b2[...]


def _tc_layer1(aggp, degt, x, W1l, b1, W1r, W2l, b2, W2r):
    grid = (N // BN,)
    return pl.pallas_call(
        _tc1_body,
        grid=grid,
        in_specs=[
            pl.BlockSpec((NC, BN, D), lambda i: (0, i, 0)),
            pl.BlockSpec((BN, NC), lambda i: (i, 0)),
            pl.BlockSpec((BN, D), lambda i: (i, 0)),
            pl.BlockSpec((D, D), lambda i: (0, 0)),
            pl.BlockSpec((D, D), lambda i: (0, 0)),
            pl.BlockSpec((1, D), lambda i: (0, 0)),
            pl.BlockSpec((C, D), lambda i: (0, 0)),
            pl.BlockSpec((C, D), lambda i: (0, 0)),
            pl.BlockSpec((1, C), lambda i: (0, 0)),
        ],
        out_specs=[
            pl.BlockSpec((BN, C), lambda i: (i, 0)),
            pl.BlockSpec((BN, C), lambda i: (i, 0)),
        ],
        out_shape=[jax.ShapeDtypeStruct((N, C), jnp.float32),
                   jax.ShapeDtypeStruct((N, C), jnp.float32)],
    )(aggp, degt, x, W1l, W1r, b1.reshape(1, D), W2l, W2r, b2.reshape(1, C))


def _tc2_body(aggp, degt, q, out):
    agg = aggp[0] + aggp[1]
    deg = degt[:, 0:1] + degt[:, 1:2]
    rdeg = 1.0 / jnp.maximum(deg, 1.0)
    z = agg * rdeg + q[...]
    m = jnp.max(z, axis=1, keepdims=True)
    zs = z - m
    out[...] = zs - jnp.log(jnp.sum(jnp.exp(zs), axis=1, keepdims=True))


def _tc_layer2(aggp, degt, q):
    grid = (N // BN,)
    return pl.pallas_call(
        _tc2_body,
        grid=grid,
        in_specs=[
            pl.BlockSpec((NC, BN, C), lambda i: (0, i, 0)),
            pl.BlockSpec((BN, NC), lambda i: (i, 0)),
            pl.BlockSpec((BN, C), lambda i: (i, 0)),
        ],
        out_specs=pl.BlockSpec((BN, C), lambda i: (i, 0)),
        out_shape=jax.ShapeDtypeStruct((N, C), jnp.float32),
    )(aggp, degt, q)


def kernel(x, edge_index, W1l, b1, W1r, W2l, b2, W2r):
    srcb = edge_index[0].reshape(NW, NG, G, K)
    dstb = edge_index[1].reshape(NW, NG, G, K)

    agg1, degp = _sc_agg_deg(x, srcb, dstb)
    degt = degp.T
    p, q = _tc_layer1(agg1, degt, x, W1l, b1, W1r, W2l, b2, W2r)
    agg2 = _sc_agg16(p, srcb, dstb)
    return _tc_layer2(agg2, degt, q)


# speedup vs baseline: 1.1498x; 1.0380x over previous
import functools

import jax
import jax.numpy as jnp
from jax import lax
from jax.experimental import pallas as pl
from jax.experimental.pallas import tpu as pltpu
from jax.experimental.pallas import tpu_sc as plsc

N = 10000
E = 320000
D = 128
C = 16

NC = 2
NS = 16
NW = NC * NS
EPW = E // NW
K = 125
NCHUNK = EPW // K
NG = 4
G = NCHUNK // NG
N2 = 10240
N2PT = N2 // NS


def _sc_mesh():
    return plsc.VectorSubcoreMesh(core_axis_name="c", subcore_axis_name="s")


def _agg_deg_body(feat, srcb, dstb, z2, z1,
                  agg_out, deg_out,
                  src_v, dst_v, rows0, rows1, ones_v, sem0, sem1,
                  acc, dega):
    c = lax.axis_index("c")
    s = lax.axis_index("s")
    w = c * NS + s

    pltpu.sync_copy(z2.at[pl.ds(s * N2PT, N2PT)], acc.at[pl.ds(s * N2PT, N2PT)])
    pltpu.sync_copy(z1.at[pl.ds(s * N2PT, N2PT)], dega.at[pl.ds(s * N2PT, N2PT)])

    for j in range(K // 16):
        ones_v[pl.ds(j * 16, 16)] = jnp.ones((16,), jnp.float32)
    if K % 16:
        ones_v[pl.ds(K - 16, 16)] = jnp.ones((16,), jnp.float32)

    plsc.subcore_barrier()

    def group(g, carry):
        pltpu.sync_copy(srcb.at[w, g], src_v)
        pltpu.sync_copy(dstb.at[w, g], dst_v)
        pltpu.async_copy(feat.at[src_v.at[0]], rows0, sem0)

        def pair(j, c2):
            i0 = 2 * j
            i1 = i0 + 1
            pltpu.async_copy(feat.at[src_v.at[i1]], rows1, sem1)
            pltpu.make_async_copy(feat.at[src_v.at[i0]], rows0, sem0).wait()
            pltpu.sync_copy(rows0, acc.at[dst_v.at[i0]], add=True)
            pltpu.sync_copy(ones_v, dega.at[dst_v.at[i0]], add=True)
            pltpu.async_copy(feat.at[src_v.at[i0 + 2]], rows0, sem0)
            pltpu.make_async_copy(feat.at[src_v.at[i1]], rows1, sem1).wait()
            pltpu.sync_copy(rows1, acc.at[dst_v.at[i1]], add=True)
            pltpu.sync_copy(ones_v, dega.at[dst_v.at[i1]], add=True)
            return c2

        lax.fori_loop(0, (G - 2) // 2, pair, 0)

        pltpu.async_copy(feat.at[src_v.at[G - 1]], rows1, sem1)
        pltpu.make_async_copy(feat.at[src_v.at[G - 2]], rows0, sem0).wait()
        pltpu.sync_copy(rows0, acc.at[dst_v.at[G - 2]], add=True)
        pltpu.sync_copy(ones_v, dega.at[dst_v.at[G - 2]], add=True)
        pltpu.make_async_copy(feat.at[src_v.at[G - 1]], rows1, sem1).wait()
        pltpu.sync_copy(rows1, acc.at[dst_v.at[G - 1]], add=True)
        pltpu.sync_copy(ones_v, dega.at[dst_v.at[G - 1]], add=True)
        return carry

    lax.fori_loop(0, NG, group, 0)

    plsc.subcore_barrier()

    pltpu.sync_copy(acc.at[pl.ds(s * N2PT, N2PT)], agg_out.at[c, pl.ds(s * N2PT, N2PT)])
    pltpu.sync_copy(dega.at[pl.ds(s * N2PT, N2PT)], deg_out.at[c, pl.ds(s * N2PT, N2PT)])


def _agg16_body(feat, srcb, dstb, z2,
                agg_out,
                src_v, dst_v, rows0, rows1, sem0, sem1,
                acc):
    c = lax.axis_index("c")
    s = lax.axis_index("s")
    w = c * NS + s

    pltpu.sync_copy(z2.at[pl.ds(s * N2PT, N2PT)], acc.at[pl.ds(s * N2PT, N2PT)])

    plsc.subcore_barrier()

    def group(g, carry):
        pltpu.sync_copy(srcb.at[w, g], src_v)
        pltpu.sync_copy(dstb.at[w, g], dst_v)
        pltpu.async_copy(feat.at[src_v.at[0]], rows0, sem0)

        def pair(j, c2):
            i0 = 2 * j
            i1 = i0 + 1
            pltpu.async_copy(feat.at[src_v.at[i1]], rows1, sem1)
            pltpu.make_async_copy(feat.at[src_v.at[i0]], rows0, sem0).wait()
            pltpu.sync_copy(rows0, acc.at[dst_v.at[i0]], add=True)
            pltpu.async_copy(feat.at[src_v.at[i0 + 2]], rows0, sem0)
            pltpu.make_async_copy(feat.at[src_v.at[i1]], rows1, sem1).wait()
            pltpu.sync_copy(rows1, acc.at[dst_v.at[i1]], add=True)
            return c2

        lax.fori_loop(0, (G - 2) // 2, pair, 0)

        pltpu.async_copy(feat.at[src_v.at[G - 1]], rows1, sem1)
        pltpu.make_async_copy(feat.at[src_v.at[G - 2]], rows0, sem0).wait()
        pltpu.sync_copy(rows0, acc.at[dst_v.at[G - 2]], add=True)
        pltpu.make_async_copy(feat.at[src_v.at[G - 1]], rows1, sem1).wait()
        pltpu.sync_copy(rows1, acc.at[dst_v.at[G - 1]], add=True)
        return carry

    lax.fori_loop(0, NG, group, 0)

    plsc.subcore_barrier()

    pltpu.sync_copy(acc.at[pl.ds(s * N2PT, N2PT)], agg_out.at[c, pl.ds(s * N2PT, N2PT)])


def _sc_agg_deg(x, srcb, dstb):
    f = pl.kernel(
        _agg_deg_body,
        out_type=[jax.ShapeDtypeStruct((NC, N2, D), jnp.float32),
                  jax.ShapeDtypeStruct((NC, N2), jnp.float32)],
        mesh=_sc_mesh(),
        scratch_types=[
            pltpu.VMEM((G, K), jnp.int32),
            pltpu.VMEM((G, K), jnp.int32),
            pltpu.VMEM((K, D), jnp.float32),
            pltpu.VMEM((K, D), jnp.float32),
            pltpu.VMEM((K,), jnp.float32),
            pltpu.SemaphoreType.DMA,
            pltpu.SemaphoreType.DMA,
            pltpu.VMEM_SHARED((N2, D), jnp.float32),
            pltpu.VMEM_SHARED((N2,), jnp.float32),
        ],
    )
    z2 = jnp.zeros((N2, D), jnp.float32)
    z1 = jnp.zeros((N2,), jnp.float32)
    return f(x, srcb, dstb, z2, z1)


def _sc_agg16(p, srcb, dstb):
    f = pl.kernel(
        _agg16_body,
        out_type=jax.ShapeDtypeStruct((NC, N2, C), jnp.float32),
        mesh=_sc_mesh(),
        compiler_params=pltpu.CompilerParams(use_tc_tiling_on_sc=False),
        scratch_types=[
            pltpu.VMEM((G, K), jnp.int32),
            pltpu.VMEM((G, K), jnp.int32),
            pltpu.VMEM((K, C), jnp.float32),
            pltpu.VMEM((K, C), jnp.float32),
            pltpu.SemaphoreType.DMA,
            pltpu.SemaphoreType.DMA,
            pltpu.VMEM_SHARED((N2, C), jnp.float32),
        ],
    )
    z2 = jnp.zeros((N2, C), jnp.float32)
    return f(p, srcb, dstb, z2)


BN = 400
_PREC = lax.Precision.HIGHEST
_DN = (((1,), (1,)), ((), ()))


def _tc1_body(aggp, degt, x, W1l, W1r, b1, W2l, W2r, b2, p_out, q_out):
    agg = aggp[0] + aggp[1]
    deg = degt[:, 0:1] + degt[:, 1:2]
    rdeg = 1.0 / jnp.maximum(deg, 1.0)
    mean = agg * rdeg
    h = (lax.dot_general(mean, W1l[...], _DN, precision=_PREC)
         + lax.dot_general(x[...], W1r[...], _DN, precision=_PREC)
         + b1[...])
    h = jnp.maximum(h, 0.0)
    p_out[...] = lax.dot_general(h, W2l[...], _DN, precision=_PREC)
    q_out[...] = lax.dot_general(h, W2r[...], _DN, precision=_PREC) + b2[...]


def _tc_layer1(aggp, degt, x, W1l, b1, W1r, W2l, b2, W2r):
    grid = (N // BN,)
    return pl.pallas_call(
        _tc1_body,
        grid=grid,
        in_specs=[
            pl.BlockSpec((NC, BN, D), lambda i: (0, i, 0)),
            pl.BlockSpec((BN, NC), lambda i: (i, 0)),
            pl.BlockSpec((BN, D), lambda i: (i, 0)),
            pl.BlockSpec((D, D), lambda i: (0, 0)),
            pl.BlockSpec((D, D), lambda i: (0, 0)),
            pl.BlockSpec((1, D), lambda i: (0, 0)),
            pl.BlockSpec((C, D), lambda i: (0, 0)),
            pl.BlockSpec((C, D), lambda i: (0, 0)),
            pl.BlockSpec((1, C), lambda i: (0, 0)),
        ],
        out_specs=[
            pl.BlockSpec((BN, C), lambda i: (i, 0)),
            pl.BlockSpec((BN, C), lambda i: (i, 0)),
        ],
        out_shape=[jax.ShapeDtypeStruct((N, C), jnp.float32),
                   jax.ShapeDtypeStruct((N, C), jnp.float32)],
    )(aggp, degt, x, W1l, W1r, b1.reshape(1, D), W2l, W2r, b2.reshape(1, C))


def _tc2_body(aggp, degt, q, out):
    agg = aggp[0] + aggp[1]
    deg = degt[:, 0:1] + degt[:, 1:2]
    rdeg = 1.0 / jnp.maximum(deg, 1.0)
    z = agg * rdeg + q[...]
    m = jnp.max(z, axis=1, keepdims=True)
    zs = z - m
    out[...] = zs - jnp.log(jnp.sum(jnp.exp(zs), axis=1, keepdims=True))


def _tc_layer2(aggp, degt, q):
    grid = (N // BN,)
    return pl.pallas_call(
        _tc2_body,
        grid=grid,
        in_specs=[
            pl.BlockSpec((NC, BN, C), lambda i: (0, i, 0)),
            pl.BlockSpec((BN, NC), lambda i: (i, 0)),
            pl.BlockSpec((BN, C), lambda i: (i, 0)),
        ],
        out_specs=pl.BlockSpec((BN, C), lambda i: (i, 0)),
        out_shape=jax.ShapeDtypeStruct((N, C), jnp.float32),
    )(aggp, degt, q)


def kernel(x, edge_index, W1l, b1, W1r, W2l, b2, W2r):
    srcb = edge_index[0].reshape(NW, NG, G, K)
    dstb = edge_index[1].reshape(NW, NG, G, K)

    agg1, degp = _sc_agg_deg(x, srcb, dstb)
    degt = degp.T
    p, q = _tc_layer1(agg1, degt, x, W1l, b1, W1r, W2l, b2, W2r)
    agg2 = _sc_agg16(p, srcb, dstb)
    return _tc_layer2(agg2, degt, q)


# speedup vs baseline: 1.1574x; 1.0067x over previous
import functools

import jax
import jax.numpy as jnp
from jax import lax
from jax.experimental import pallas as pl
from jax.experimental.pallas import tpu as pltpu
from jax.experimental.pallas import tpu_sc as plsc

N = 10000
E = 320000
D = 128
C = 16

NC = 2
NS = 16
NW = NC * NS
EPW = E // NW
K = 125
NCHUNK = EPW // K
NG = 4
G = NCHUNK // NG
N2 = 10240
N2PT = N2 // NS


def _sc_mesh():
    return plsc.VectorSubcoreMesh(core_axis_name="c", subcore_axis_name="s")


def _agg_deg_body(feat, srcb, dstb, z2, z1,
                  agg_out, deg_out,
                  src_v, dst_v, rows0, rows1, ones_v, sem0, sem1, dsem,
                  acc, dega):
    c = lax.axis_index("c")
    s = lax.axis_index("s")
    w = c * NS + s

    pltpu.sync_copy(z2.at[pl.ds(s * N2PT, N2PT)], acc.at[pl.ds(s * N2PT, N2PT)])
    pltpu.sync_copy(z1.at[pl.ds(s * N2PT, N2PT)], dega.at[pl.ds(s * N2PT, N2PT)])

    for j in range(K // 16):
        ones_v[pl.ds(j * 16, 16)] = jnp.ones((16,), jnp.float32)
    if K % 16:
        ones_v[pl.ds(K - 16, 16)] = jnp.ones((16,), jnp.float32)

    plsc.subcore_barrier()

    def group(g, carry):
        pltpu.sync_copy(srcb.at[w, g], src_v)
        pltpu.sync_copy(dstb.at[w, g], dst_v)
        pltpu.async_copy(feat.at[src_v.at[0]], rows0, sem0)

        def pair(j, c2):
            i0 = 2 * j
            i1 = i0 + 1
            pltpu.async_copy(feat.at[src_v.at[i1]], rows1, sem1)
            pltpu.make_async_copy(feat.at[src_v.at[i0]], rows0, sem0).wait()
            pltpu.sync_copy(rows0, acc.at[dst_v.at[i0]], add=True)
            pltpu.async_copy(ones_v, dega.at[dst_v.at[i0]], dsem, add=True)
            pltpu.async_copy(feat.at[src_v.at[i0 + 2]], rows0, sem0)
            pltpu.make_async_copy(feat.at[src_v.at[i1]], rows1, sem1).wait()
            pltpu.sync_copy(rows1, acc.at[dst_v.at[i1]], add=True)
            pltpu.async_copy(ones_v, dega.at[dst_v.at[i1]], dsem, add=True)
            return c2

        lax.fori_loop(0, (G - 2) // 2, pair, 0)

        pltpu.async_copy(feat.at[src_v.at[G - 1]], rows1, sem1)
        pltpu.make_async_copy(feat.at[src_v.at[G - 2]], rows0, sem0).wait()
        pltpu.sync_copy(rows0, acc.at[dst_v.at[G - 2]], add=True)
        pltpu.async_copy(ones_v, dega.at[dst_v.at[G - 2]], dsem, add=True)
        pltpu.make_async_copy(feat.at[src_v.at[G - 1]], rows1, sem1).wait()
        pltpu.sync_copy(rows1, acc.at[dst_v.at[G - 1]], add=True)
        pltpu.async_copy(ones_v, dega.at[dst_v.at[G - 1]], dsem, add=True)

        def drain(i, c2):
            pltpu.make_async_copy(ones_v, dega.at[dst_v.at[0]], dsem).wait()
            return c2

        lax.fori_loop(0, G, drain, 0)
        return carry

    lax.fori_loop(0, NG, group, 0)

    plsc.subcore_barrier()

    pltpu.sync_copy(acc.at[pl.ds(s * N2PT, N2PT)], agg_out.at[c, pl.ds(s * N2PT, N2PT)])
    pltpu.sync_copy(dega.at[pl.ds(s * N2PT, N2PT)], deg_out.at[c, pl.ds(s * N2PT, N2PT)])


def _agg16_body(feat, srcb, dstb, z2,
                agg_out,
                src_v, dst_v, rows0, rows1, sem0, sem1,
                acc):
    c = lax.axis_index("c")
    s = lax.axis_index("s")
    w = c * NS + s

    pltpu.sync_copy(z2.at[pl.ds(s * N2PT, N2PT)], acc.at[pl.ds(s * N2PT, N2PT)])

    plsc.subcore_barrier()

    def group(g, carry):
        pltpu.sync_copy(srcb.at[w, g], src_v)
        pltpu.sync_copy(dstb.at[w, g], dst_v)
        pltpu.async_copy(feat.at[src_v.at[0]], rows0, sem0)

        def pair(j, c2):
            i0 = 2 * j
            i1 = i0 + 1
            pltpu.async_copy(feat.at[src_v.at[i1]], rows1, sem1)
            pltpu.make_async_copy(feat.at[src_v.at[i0]], rows0, sem0).wait()
            pltpu.sync_copy(rows0, acc.at[dst_v.at[i0]], add=True)
            pltpu.async_copy(feat.at[src_v.at[i0 + 2]], rows0, sem0)
            pltpu.make_async_copy(feat.at[src_v.at[i1]], rows1, sem1).wait()
            pltpu.sync_copy(rows1, acc.at[dst_v.at[i1]], add=True)
            return c2

        lax.fori_loop(0, (G - 2) // 2, pair, 0)

        pltpu.async_copy(feat.at[src_v.at[G - 1]], rows1, sem1)
        pltpu.make_async_copy(feat.at[src_v.at[G - 2]], rows0, sem0).wait()
        pltpu.sync_copy(rows0, acc.at[dst_v.at[G - 2]], add=True)
        pltpu.make_async_copy(feat.at[src_v.at[G - 1]], rows1, sem1).wait()
        pltpu.sync_copy(rows1, acc.at[dst_v.at[G - 1]], add=True)
        return carry

    lax.fori_loop(0, NG, group, 0)

    plsc.subcore_barrier()

    pltpu.sync_copy(acc.at[pl.ds(s * N2PT, N2PT)], agg_out.at[c, pl.ds(s * N2PT, N2PT)])


def _sc_agg_deg(x, srcb, dstb):
    f = pl.kernel(
        _agg_deg_body,
        out_type=[jax.ShapeDtypeStruct((NC, N2, D), jnp.float32),
                  jax.ShapeDtypeStruct((NC, N2), jnp.float32)],
        mesh=_sc_mesh(),
        scratch_types=[
            pltpu.VMEM((G, K), jnp.int32),
            pltpu.VMEM((G, K), jnp.int32),
            pltpu.VMEM((K, D), jnp.float32),
            pltpu.VMEM((K, D), jnp.float32),
            pltpu.VMEM((K,), jnp.float32),
            pltpu.SemaphoreType.DMA,
            pltpu.SemaphoreType.DMA,
            pltpu.SemaphoreType.DMA,
            pltpu.VMEM_SHARED((N2, D), jnp.float32),
            pltpu.VMEM_SHARED((N2,), jnp.float32),
        ],
    )
    z2 = jnp.zeros((N2, D), jnp.float32)
    z1 = jnp.zeros((N2,), jnp.float32)
    return f(x, srcb, dstb, z2, z1)


def _sc_agg16(p, srcb, dstb):
    f = pl.kernel(
        _agg16_body,
        out_type=jax.ShapeDtypeStruct((NC, N2, C), jnp.float32),
        mesh=_sc_mesh(),
        compiler_params=pltpu.CompilerParams(use_tc_tiling_on_sc=False),
        scratch_types=[
            pltpu.VMEM((G, K), jnp.int32),
            pltpu.VMEM((G, K), jnp.int32),
            pltpu.VMEM((K, C), jnp.float32),
            pltpu.VMEM((K, C), jnp.float32),
            pltpu.SemaphoreType.DMA,
            pltpu.SemaphoreType.DMA,
            pltpu.VMEM_SHARED((N2, C), jnp.float32),
        ],
    )
    z2 = jnp.zeros((N2, C), jnp.float32)
    return f(p, srcb, dstb, z2)


BN = 400
_PREC = lax.Precision.HIGHEST
_DN = (((1,), (1,)), ((), ()))


def _tc1_body(aggp, degt, x, W1l, W1r, b1, W2l, W2r, b2, p_out, q_out):
    agg = aggp[0] + aggp[1]
    deg = degt[:, 0:1] + degt[:, 1:2]
    rdeg = 1.0 / jnp.maximum(deg, 1.0)
    mean = agg * rdeg
    h = (lax.dot_general(mean, W1l[...], _DN, precision=_PREC)
         + lax.dot_general(x[...], W1r[...], _DN, precision=_PREC)
         + b1[...])
    h = jnp.maximum(h, 0.0)
    p_out[...] = lax.dot_general(h, W2l[...], _DN, precision=_PREC)
    q_out[...] = lax.dot_general(h, W2r[...], _DN, precision=_PREC) + b2[...]


def _tc_layer1(aggp, degt, x, W1l, b1, W1r, W2l, b2, W2r):
    grid = (N // BN,)
    return pl.pallas_call(
        _tc1_body,
        grid=grid,
        in_specs=[
            pl.BlockSpec((NC, BN, D), lambda i: (0, i, 0)),
            pl.BlockSpec((BN, NC), lambda i: (i, 0)),
            pl.BlockSpec((BN, D), lambda i: (i, 0)),
            pl.BlockSpec((D, D), lambda i: (0, 0)),
            pl.BlockSpec((D, D), lambda i: (0, 0)),
            pl.BlockSpec((1, D), lambda i: (0, 0)),
            pl.BlockSpec((C, D), lambda i: (0, 0)),
            pl.BlockSpec((C, D), lambda i: (0, 0)),
            pl.BlockSpec((1, C), lambda i: (0, 0)),
        ],
        out_specs=[
            pl.BlockSpec((BN, C), lambda i: (i, 0)),
            pl.BlockSpec((BN, C), lambda i: (i, 0)),
        ],
        out_shape=[jax.ShapeDtypeStruct((N, C), jnp.float32),
                   jax.ShapeDtypeStruct((N, C), jnp.float32)],
    )(aggp, degt, x, W1l, W1r, b1.reshape(1, D), W2l, W2r, b2.reshape(1, C))


def _tc2_body(aggp, degt, q, out):
    agg = aggp[0] + aggp[1]
    deg = degt[:, 0:1] + degt[:, 1:2]
    rdeg = 1.0 / jnp.maximum(deg, 1.0)
    z = agg * rdeg + q[...]
    m = jnp.max(z, axis=1, keepdims=True)
    zs = z - m
    out[...] = zs - jnp.log(jnp.sum(jnp.exp(zs), axis=1, keepdims=True))


def _tc_layer2(aggp, degt, q):
    grid = (N // BN,)
    return pl.pallas_call(
        _tc2_body,
        grid=grid,
        in_specs=[
            pl.BlockSpec((NC, BN, C), lambda i: (0, i, 0)),
            pl.BlockSpec((BN, NC), lambda i: (i, 0)),
            pl.BlockSpec((BN, C), lambda i: (i, 0)),
        ],
        out_specs=pl.BlockSpec((BN, C), lambda i: (i, 0)),
        out_shape=jax.ShapeDtypeStruct((N, C), jnp.float32),
    )(aggp, degt, q)


def kernel(x, edge_index, W1l, b1, W1r, W2l, b2, W2r):
    srcb = edge_index[0].reshape(NW, NG, G, K)
    dstb = edge_index[1].reshape(NW, NG, G, K)

    agg1, degp = _sc_agg_deg(x, srcb, dstb)
    degt = degp.T
    p, q = _tc_layer1(agg1, degt, x, W1l, b1, W1r, W2l, b2, W2r)
    agg2 = _sc_agg16(p, srcb, dstb)
    return _tc_layer2(agg2, degt, q)


# speedup vs baseline: 1.1975x; 1.0346x over previous
import functools

import jax
import jax.numpy as jnp
from jax import lax
from jax.experimental import pallas as pl
from jax.experimental.pallas import tpu as pltpu
from jax.experimental.pallas import tpu_sc as plsc

N = 10000
E = 320000
D = 128
C = 16

NC = 2
NS = 16
NW = NC * NS
EPW = E // NW
K = 125
NCHUNK = EPW // K
NG = 4
G = NCHUNK // NG
N2 = 10240
N2PT = N2 // NS


def _sc_mesh():
    return plsc.VectorSubcoreMesh(core_axis_name="c", subcore_axis_name="s")


def _agg_deg_body(feat, eidx, z2, z1,
                  agg_out, deg_out,
                  src_v, dst_v, rows0, rows1, ones_v, sem0, sem1, dsem,
                  acc, dega):
    c = lax.axis_index("c")
    s = lax.axis_index("s")
    w = c * NS + s

    pltpu.sync_copy(z2.at[pl.ds(s * N2PT, N2PT)], acc.at[pl.ds(s * N2PT, N2PT)])
    pltpu.sync_copy(z1.at[pl.ds(s * N2PT, N2PT)], dega.at[pl.ds(s * N2PT, N2PT)])

    for j in range(K // 16):
        ones_v[pl.ds(j * 16, 16)] = jnp.ones((16,), jnp.float32)
    if K % 16:
        ones_v[pl.ds(K - 16, 16)] = jnp.ones((16,), jnp.float32)

    plsc.subcore_barrier()

    def group(g, carry):
        pltpu.sync_copy(eidx.at[0, w, g], src_v)
        pltpu.sync_copy(eidx.at[1, w, g], dst_v)
        pltpu.async_copy(feat.at[src_v.at[0]], rows0, sem0)

        def pair(j, c2):
            i0 = 2 * j
            i1 = i0 + 1
            pltpu.async_copy(feat.at[src_v.at[i1]], rows1, sem1)
            pltpu.make_async_copy(feat.at[src_v.at[i0]], rows0, sem0).wait()
            pltpu.sync_copy(rows0, acc.at[dst_v.at[i0]], add=True)
            pltpu.async_copy(ones_v, dega.at[dst_v.at[i0]], dsem, add=True)
            pltpu.async_copy(feat.at[src_v.at[i0 + 2]], rows0, sem0)
            pltpu.make_async_copy(feat.at[src_v.at[i1]], rows1, sem1).wait()
            pltpu.sync_copy(rows1, acc.at[dst_v.at[i1]], add=True)
            pltpu.async_copy(ones_v, dega.at[dst_v.at[i1]], dsem, add=True)
            return c2

        lax.fori_loop(0, (G - 2) // 2, pair, 0)

        pltpu.async_copy(feat.at[src_v.at[G - 1]], rows1, sem1)
        pltpu.make_async_copy(feat.at[src_v.at[G - 2]], rows0, sem0).wait()
        pltpu.sync_copy(rows0, acc.at[dst_v.at[G - 2]], add=True)
        pltpu.async_copy(ones_v, dega.at[dst_v.at[G - 2]], dsem, add=True)
        pltpu.make_async_copy(feat.at[src_v.at[G - 1]], rows1, sem1).wait()
        pltpu.sync_copy(rows1, acc.at[dst_v.at[G - 1]], add=True)
        pltpu.async_copy(ones_v, dega.at[dst_v.at[G - 1]], dsem, add=True)

        def drain(i, c2):
            pltpu.make_async_copy(ones_v, dega.at[dst_v.at[0]], dsem).wait()
            return c2

        lax.fori_loop(0, G, drain, 0)
        return carry

    lax.fori_loop(0, NG, group, 0)

    plsc.subcore_barrier()

    pltpu.sync_copy(acc.at[pl.ds(s * N2PT, N2PT)], agg_out.at[c, pl.ds(s * N2PT, N2PT)])
    pltpu.sync_copy(dega.at[pl.ds(s * N2PT, N2PT)], deg_out.at[c, pl.ds(s * N2PT, N2PT)])


def _agg16_body(feat, eidx, z2,
                agg_out,
                src_v, dst_v, rows0, rows1, sem0, sem1,
                acc):
    c = lax.axis_index("c")
    s = lax.axis_index("s")
    w = c * NS + s

    pltpu.sync_copy(z2.at[pl.ds(s * N2PT, N2PT)], acc.at[pl.ds(s * N2PT, N2PT)])

    plsc.subcore_barrier()

    def group(g, carry):
        pltpu.sync_copy(eidx.at[0, w, g], src_v)
        pltpu.sync_copy(eidx.at[1, w, g], dst_v)
        pltpu.async_copy(feat.at[src_v.at[0]], rows0, sem0)

        def pair(j, c2):
            i0 = 2 * j
            i1 = i0 + 1
            pltpu.async_copy(feat.at[src_v.at[i1]], rows1, sem1)
            pltpu.make_async_copy(feat.at[src_v.at[i0]], rows0, sem0).wait()
            pltpu.sync_copy(rows0, acc.at[dst_v.at[i0]], add=True)
            pltpu.async_copy(feat.at[src_v.at[i0 + 2]], rows0, sem0)
            pltpu.make_async_copy(feat.at[src_v.at[i1]], rows1, sem1).wait()
            pltpu.sync_copy(rows1, acc.at[dst_v.at[i1]], add=True)
            return c2

        lax.fori_loop(0, (G - 2) // 2, pair, 0)

        pltpu.async_copy(feat.at[src_v.at[G - 1]], rows1, sem1)
        pltpu.make_async_copy(feat.at[src_v.at[G - 2]], rows0, sem0).wait()
        pltpu.sync_copy(rows0, acc.at[dst_v.at[G - 2]], add=True)
        pltpu.make_async_copy(feat.at[src_v.at[G - 1]], rows1, sem1).wait()
        pltpu.sync_copy(rows1, acc.at[dst_v.at[G - 1]], add=True)
        return carry

    lax.fori_loop(0, NG, group, 0)

    plsc.subcore_barrier()

    pltpu.sync_copy(acc.at[pl.ds(s * N2PT, N2PT)], agg_out.at[c, pl.ds(s * N2PT, N2PT)])


def _sc_agg_deg(x, eidx):
    f = pl.kernel(
        _agg_deg_body,
        out_type=[jax.ShapeDtypeStruct((NC, N2, D), jnp.float32),
                  jax.ShapeDtypeStruct((NC, N2), jnp.float32)],
        mesh=_sc_mesh(),
        scratch_types=[
            pltpu.VMEM((G, K), jnp.int32),
            pltpu.VMEM((G, K), jnp.int32),
            pltpu.VMEM((K, D), jnp.float32),
            pltpu.VMEM((K, D), jnp.float32),
            pltpu.VMEM((K,), jnp.float32),
            pltpu.SemaphoreType.DMA,
            pltpu.SemaphoreType.DMA,
            pltpu.SemaphoreType.DMA,
            pltpu.VMEM_SHARED((N2, D), jnp.float32),
            pltpu.VMEM_SHARED((N2,), jnp.float32),
        ],
    )
    z2 = jnp.zeros((N2, D), jnp.float32)
    z1 = jnp.zeros((N2,), jnp.float32)
    return f(x, eidx, z2, z1)


def _sc_agg16(p, eidx):
    f = pl.kernel(
        _agg16_body,
        out_type=jax.ShapeDtypeStruct((NC, N2, C), jnp.float32),
        mesh=_sc_mesh(),
        compiler_params=pltpu.CompilerParams(use_tc_tiling_on_sc=False),
        scratch_types=[
            pltpu.VMEM((G, K), jnp.int32),
            pltpu.VMEM((G, K), jnp.int32),
            pltpu.VMEM((K, C), jnp.float32),
            pltpu.VMEM((K, C), jnp.float32),
            pltpu.SemaphoreType.DMA,
            pltpu.SemaphoreType.DMA,
            pltpu.VMEM_SHARED((N2, C), jnp.float32),
        ],
    )
    z2 = jnp.zeros((N2, C), jnp.float32)
    return f(p, eidx, z2)


BN = 400
_PREC = lax.Precision.HIGHEST
_DN = (((1,), (1,)), ((), ()))


def _tc1_body(aggp, degt, x, W1l, W1r, b1, W2l, W2r, b2, p_out, q_out):
    agg = aggp[0] + aggp[1]
    deg = degt[:, 0:1] + degt[:, 1:2]
    rdeg = 1.0 / jnp.maximum(deg, 1.0)
    mean = agg * rdeg
    h = (lax.dot_general(mean, W1l[...], _DN, precision=_PREC)
         + lax.dot_general(x[...], W1r[...], _DN, precision=_PREC)
         + b1[...])
    h = jnp.maximum(h, 0.0)
    p_out[...] = lax.dot_general(h, W2l[...], _DN, precision=_PREC)
    q_out[...] = lax.dot_general(h, W2r[...], _DN, precision=_PREC) + b2[...]


def _tc_layer1(aggp, degp, x, W1l, b1, W1r, W2l, b2, W2r):
    grid = (N // BN,)
    return pl.pallas_call(
        _tc1_body,
        grid=grid,
        in_specs=[
            pl.BlockSpec((NC, BN, D), lambda i: (0, i, 0)),
            pl.BlockSpec((BN, NC), lambda i: (i, 0)),
            pl.BlockSpec((BN, D), lambda i: (i, 0)),
            pl.BlockSpec((D, D), lambda i: (0, 0)),
            pl.BlockSpec((D, D), lambda i: (0, 0)),
            pl.BlockSpec((1, D), lambda i: (0, 0)),
            pl.BlockSpec((C, D), lambda i: (0, 0)),
            pl.BlockSpec((C, D), lambda i: (0, 0)),
            pl.BlockSpec((1, C), lambda i: (0, 0)),
        ],
        out_specs=[
            pl.BlockSpec((BN, C), lambda i: (i, 0)),
            pl.BlockSpec((BN, C), lambda i: (i, 0)),
        ],
        out_shape=[jax.ShapeDtypeStruct((N, C), jnp.float32),
                   jax.ShapeDtypeStruct((N, C), jnp.float32)],
    )(aggp, degp, x, W1l, W1r, b1.reshape(1, D), W2l, W2r, b2.reshape(1, C))


def _tc2_body(aggp, degt, q, out):
    agg = aggp[0] + aggp[1]
    deg = degt[:, 0:1] + degt[:, 1:2]
    rdeg = 1.0 / jnp.maximum(deg, 1.0)
    z = agg * rdeg + q[...]
    m = jnp.max(z, axis=1, keepdims=True)
    zs = z - m
    out[...] = zs - jnp.log(jnp.sum(jnp.exp(zs), axis=1, keepdims=True))


def _tc_layer2(aggp, degp, q):
    grid = (N // BN,)
    return pl.pallas_call(
        _tc2_body,
        grid=grid,
        in_specs=[
            pl.BlockSpec((NC, BN, C), lambda i: (0, i, 0)),
            pl.BlockSpec((BN, NC), lambda i: (i, 0)),
            pl.BlockSpec((BN, C), lambda i: (i, 0)),
        ],
        out_specs=pl.BlockSpec((BN, C), lambda i: (i, 0)),
        out_shape=jax.ShapeDtypeStruct((N, C), jnp.float32),
    )(aggp, degp, q)


def kernel(x, edge_index, W1l, b1, W1r, W2l, b2, W2r):
    eidx = edge_index.reshape(2, NW, NG, G, K)

    agg1, degp = _sc_agg_deg(x, eidx)
    degt = degp.T
    p, q = _tc_layer1(agg1, degt, x, W1l, b1, W1r, W2l, b2, W2r)
    agg2 = _sc_agg16(p, eidx)
    return _tc_layer2(agg2, degt, q)


# speedup vs baseline: 1.2317x; 1.0286x over previous
import jax
import jax.numpy as jnp
from jax import lax
from jax.experimental import pallas as pl
from jax.experimental.pallas import tpu as pltpu
from jax.experimental.pallas import tpu_sc as plsc

N = 10000
E = 320000
D = 128
C = 16

NC = 2
NS = 16
NW = NC * NS
EPW = E // NW
K = 125
NCHUNK = EPW // K
NG = 4
G = NCHUNK // NG
N2 = 10240
N2PT = N2 // NS


def _sc_mesh():
    return plsc.VectorSubcoreMesh(core_axis_name="c", subcore_axis_name="s")


def _agg_deg_body(feat, eidx, z2, z1,
                  agg_out, deg_out,
                  src_v, dst_v, rows0, rows1, ones_v, sem0, sem1, dsem,
                  acc, dega):
    c = lax.axis_index("c")
    s = lax.axis_index("s")
    w = c * NS + s

    pltpu.sync_copy(z2.at[pl.ds(s * N2PT, N2PT)], acc.at[pl.ds(s * N2PT, N2PT)])
    pltpu.sync_copy(z1.at[pl.ds(s * N2PT, N2PT)], dega.at[pl.ds(s * N2PT, N2PT)])

    for j in range(K // 16):
        ones_v[pl.ds(j * 16, 16)] = jnp.ones((16,), jnp.float32)
    if K % 16:
        ones_v[pl.ds(K - 16, 16)] = jnp.ones((16,), jnp.float32)

    plsc.subcore_barrier()

    def group(g, carry):
        pltpu.sync_copy(eidx.at[0, w, g], src_v)
        pltpu.sync_copy(eidx.at[1, w, g], dst_v)
        pltpu.async_copy(feat.at[src_v.at[0]], rows0, sem0)

        def pair(j, c2):
            i0 = 2 * j
            i1 = i0 + 1
            pltpu.async_copy(feat.at[src_v.at[i1]], rows1, sem1)
            pltpu.make_async_copy(feat.at[src_v.at[i0]], rows0, sem0).wait()
            pltpu.sync_copy(rows0, acc.at[dst_v.at[i0]], add=True)
            pltpu.async_copy(ones_v, dega.at[dst_v.at[i0]], dsem, add=True)
            pltpu.async_copy(feat.at[src_v.at[i0 + 2]], rows0, sem0)
            pltpu.make_async_copy(feat.at[src_v.at[i1]], rows1, sem1).wait()
            pltpu.sync_copy(rows1, acc.at[dst_v.at[i1]], add=True)
            pltpu.async_copy(ones_v, dega.at[dst_v.at[i1]], dsem, add=True)
            return c2

        lax.fori_loop(0, (G - 2) // 2, pair, 0)

        pltpu.async_copy(feat.at[src_v.at[G - 1]], rows1, sem1)
        pltpu.make_async_copy(feat.at[src_v.at[G - 2]], rows0, sem0).wait()
        pltpu.sync_copy(rows0, acc.at[dst_v.at[G - 2]], add=True)
        pltpu.async_copy(ones_v, dega.at[dst_v.at[G - 2]], dsem, add=True)
        pltpu.make_async_copy(feat.at[src_v.at[G - 1]], rows1, sem1).wait()
        pltpu.sync_copy(rows1, acc.at[dst_v.at[G - 1]], add=True)
        pltpu.async_copy(ones_v, dega.at[dst_v.at[G - 1]], dsem, add=True)

        def drain(i, c2):
            pltpu.make_async_copy(ones_v, dega.at[dst_v.at[0]], dsem).wait()
            return c2

        lax.fori_loop(0, G, drain, 0)
        return carry

    lax.fori_loop(0, NG, group, 0)

    plsc.subcore_barrier()

    pltpu.sync_copy(acc.at[pl.ds(s * N2PT, N2PT)], agg_out.at[c, pl.ds(s * N2PT, N2PT)])
    pltpu.sync_copy(dega.at[pl.ds(s * N2PT, N2PT)], deg_out.at[c, pl.ds(s * N2PT, N2PT)])


def _agg16_body(feat, eidx, z2,
                agg_out,
                src_v, dst_v, rows0, rows1, sem0, sem1,
                acc):
    c = lax.axis_index("c")
    s = lax.axis_index("s")
    w = c * NS + s

    pltpu.sync_copy(z2.at[pl.ds(s * N2PT, N2PT)], acc.at[pl.ds(s * N2PT, N2PT)])

    plsc.subcore_barrier()

    def group(g, carry):
        pltpu.sync_copy(eidx.at[0, w, g], src_v)
        pltpu.sync_copy(eidx.at[1, w, g], dst_v)
        pltpu.async_copy(feat.at[src_v.at[0]], rows0, sem0)

        def pair(j, c2):
            i0 = 2 * j
            i1 = i0 + 1
            pltpu.async_copy(feat.at[src_v.at[i1]], rows1, sem1)
            pltpu.make_async_copy(feat.at[src_v.at[i0]], rows0, sem0).wait()
            pltpu.sync_copy(rows0, acc.at[dst_v.at[i0]], add=True)
            pltpu.async_copy(feat.at[src_v.at[i0 + 2]], rows0, sem0)
            pltpu.make_async_copy(feat.at[src_v.at[i1]], rows1, sem1).wait()
            pltpu.sync_copy(rows1, acc.at[dst_v.at[i1]], add=True)
            return c2

        lax.fori_loop(0, (G - 2) // 2, pair, 0)

        pltpu.async_copy(feat.at[src_v.at[G - 1]], rows1, sem1)
        pltpu.make_async_copy(feat.at[src_v.at[G - 2]], rows0, sem0).wait()
        pltpu.sync_copy(rows0, acc.at[dst_v.at[G - 2]], add=True)
        pltpu.make_async_copy(feat.at[src_v.at[G - 1]], rows1, sem1).wait()
        pltpu.sync_copy(rows1, acc.at[dst_v.at[G - 1]], add=True)
        return carry

    lax.fori_loop(0, NG, group, 0)

    plsc.subcore_barrier()

    pltpu.sync_copy(acc.at[pl.ds(s * N2PT, N2PT)], agg_out.at[c, pl.ds(s * N2PT, N2PT)])


def _sc_agg_deg(x, eidx):
    f = pl.kernel(
        _agg_deg_body,
        out_type=[jax.ShapeDtypeStruct((NC, N2, D), jnp.float32),
                  jax.ShapeDtypeStruct((NC, N2), jnp.float32)],
        mesh=_sc_mesh(),
        scratch_types=[
            pltpu.VMEM((G, K), jnp.int32),
            pltpu.VMEM((G, K), jnp.int32),
            pltpu.VMEM((K, D), jnp.float32),
            pltpu.VMEM((K, D), jnp.float32),
            pltpu.VMEM((K,), jnp.float32),
            pltpu.SemaphoreType.DMA,
            pltpu.SemaphoreType.DMA,
            pltpu.SemaphoreType.DMA,
            pltpu.VMEM_SHARED((N2, D), jnp.float32),
            pltpu.VMEM_SHARED((N2,), jnp.float32),
        ],
    )
    z2 = jnp.zeros((N2, D), jnp.float32)
    z1 = jnp.zeros((N2,), jnp.float32)
    return f(x, eidx, z2, z1)


def _sc_agg16(p, eidx):
    f = pl.kernel(
        _agg16_body,
        out_type=jax.ShapeDtypeStruct((NC, N2, C), jnp.float32),
        mesh=_sc_mesh(),
        compiler_params=pltpu.CompilerParams(use_tc_tiling_on_sc=False),
        scratch_types=[
            pltpu.VMEM((G, K), jnp.int32),
            pltpu.VMEM((G, K), jnp.int32),
            pltpu.VMEM((K, C), jnp.float32),
            pltpu.VMEM((K, C), jnp.float32),
            pltpu.SemaphoreType.DMA,
            pltpu.SemaphoreType.DMA,
            pltpu.VMEM_SHARED((N2, C), jnp.float32),
        ],
    )
    z2 = jnp.zeros((N2, C), jnp.float32)
    return f(p, eidx, z2)


BN = 10000
_PREC = lax.Precision.HIGHEST
_DN = (((1,), (1,)), ((), ()))


def _tc1_body(aggp, degt, x, W1l, W1r, b1, W2l, W2r, b2, p_out, q_out):
    agg = aggp[0] + aggp[1]
    deg = degt[:, 0:1] + degt[:, 1:2]
    rdeg = 1.0 / jnp.maximum(deg, 1.0)
    mean = agg * rdeg
    h = (lax.dot_general(mean, W1l[...], _DN, precision=_PREC)
         + lax.dot_general(x[...], W1r[...], _DN, precision=_PREC)
         + b1[...])
    h = jnp.maximum(h, 0.0)
    p_out[...] = lax.dot_general(h, W2l[...], _DN, precision=_PREC)
    q_out[...] = lax.dot_general(h, W2r[...], _DN, precision=_PREC) + b2[...]


def _tc_layer1(aggp, degp, x, W1l, b1, W1r, W2l, b2, W2r):
    grid = (N // BN,)
    return pl.pallas_call(
        _tc1_body,
        grid=grid,
        in_specs=[
            pl.BlockSpec((NC, BN, D), lambda i: (0, i, 0)),
            pl.BlockSpec((BN, NC), lambda i: (i, 0)),
            pl.BlockSpec((BN, D), lambda i: (i, 0)),
            pl.BlockSpec((D, D), lambda i: (0, 0)),
            pl.BlockSpec((D, D), lambda i: (0, 0)),
            pl.BlockSpec((1, D), lambda i: (0, 0)),
            pl.BlockSpec((C, D), lambda i: (0, 0)),
            pl.BlockSpec((C, D), lambda i: (0, 0)),
            pl.BlockSpec((1, C), lambda i: (0, 0)),
        ],
        out_specs=[
            pl.BlockSpec((BN, C), lambda i: (i, 0)),
            pl.BlockSpec((BN, C), lambda i: (i, 0)),
        ],
        out_shape=[jax.ShapeDtypeStruct((N, C), jnp.float32),
                   jax.ShapeDtypeStruct((N, C), jnp.float32)],
    )(aggp, degp, x, W1l, W1r, b1.reshape(1, D), W2l, W2r, b2.reshape(1, C))


def _tc2_body(aggp, degt, q, out):
    agg = aggp[0] + aggp[1]
    deg = degt[:, 0:1] + degt[:, 1:2]
    rdeg = 1.0 / jnp.maximum(deg, 1.0)
    z = agg * rdeg + q[...]
    m = jnp.max(z, axis=1, keepdims=True)
    zs = z - m
    out[...] = zs - jnp.log(jnp.sum(jnp.exp(zs), axis=1, keepdims=True))


def _tc_layer2(aggp, degp, q):
    grid = (N // BN,)
    return pl.pallas_call(
        _tc2_body,
        grid=grid,
        in_specs=[
            pl.BlockSpec((NC, BN, C), lambda i: (0, i, 0)),
            pl.BlockSpec((BN, NC), lambda i: (i, 0)),
            pl.BlockSpec((BN, C), lambda i: (i, 0)),
        ],
        out_specs=pl.BlockSpec((BN, C), lambda i: (i, 0)),
        out_shape=jax.ShapeDtypeStruct((N, C), jnp.float32),
    )(aggp, degp, q)


def kernel(x, edge_index, W1l, b1, W1r, W2l, b2, W2r):
    eidx = edge_index.reshape(2, NW, NG, G, K)

    agg1, degp = _sc_agg_deg(x, eidx)
    degt = degp.T
    p, q = _tc_layer1(agg1, degt, x, W1l, b1, W1r, W2l, b2, W2r)
    agg2 = _sc_agg16(p, eidx)
    return _tc_layer2(agg2, degt, q)
